# Initial kernel scaffold; baseline (speedup 1.0000x reference)
import jax
import jax.numpy as jnp
from jax.experimental import pallas as pl


def kernel(xyz, rots, offsets, aabbs, W1, b1, wd, Wa):
    raise NotImplementedError("write your pallas kernel here")


# speedup vs baseline: 1.9259x; 1.9259x over previous
import jax
import jax.numpy as jnp
from jax.experimental import pallas as pl
from jax.experimental.pallas import tpu as pltpu

_N_RF = 8


def _bdot(a, b, dims):
    return jax.lax.dot_general(
        a.astype(jnp.bfloat16), b.astype(jnp.bfloat16), (dims, ((), ())),
        preferred_element_type=jnp.float32)


def _fused_body(xyz_ref, rots_ref, offs_ref, W1_ref, b1_ref, wd_ref, Wa_ref,
                sigma_ref, feat_ref):
    xb = xyz_ref[...]
    best_clip = None
    sigma = None
    feat = None
    for r in range(_N_RF):
        rxyz = _bdot(xb, rots_ref[r], ((1,), (1,)))
        oxyz = rxyz + offs_ref[r]
        pre = _bdot(oxyz, W1_ref[r], ((1,), (0,)))
        h = jnp.maximum(pre + b1_ref[r][None, :], 0.0)
        sig = _bdot(h, wd_ref[r][:, None], ((1,), (0,)))
        ft = _bdot(h, Wa_ref[r], ((1,), (0,)))
        clip = jnp.clip(sig, -10.0, 10.0)
        if r == 0:
            best_clip, sigma, feat = clip, sig, ft
        else:
            upd = clip > best_clip
            best_clip = jnp.where(upd, clip, best_clip)
            sigma = jnp.where(upd, sig, sigma)
            feat = jnp.where(upd, ft, feat)
    sigma_ref[...] = sigma
    feat_ref[...] = feat


def kernel(xyz, rots, offsets, aabbs, W1, b1, wd, Wa):
    del aabbs
    n = xyz.shape[0]
    blk = 1024
    grid = (n // blk,)
    whole = lambda *dims: pl.BlockSpec(dims, lambda i: (0,) * len(dims))
    sigma2, feat = pl.pallas_call(
        _fused_body,
        grid=grid,
        in_specs=[
            pl.BlockSpec((blk, 3), lambda i: (i, 0)),
            whole(_N_RF, 3, 3),
            whole(_N_RF, 1, 3),
            whole(_N_RF, 3, 256),
            whole(_N_RF, 256),
            whole(_N_RF, 256),
            whole(_N_RF, 256, 128),
        ],
        out_specs=[
            pl.BlockSpec((blk, 1), lambda i: (i, 0)),
            pl.BlockSpec((blk, 128), lambda i: (i, 0)),
        ],
        out_shape=[
            jax.ShapeDtypeStruct((n, 1), jnp.float32),
            jax.ShapeDtypeStruct((n, 128), jnp.float32),
        ],
        compiler_params=pltpu.CompilerParams(
            dimension_semantics=("parallel",)),
    )(xyz, rots, offsets[:, :1, :3], W1, b1, wd, Wa)
    return sigma2.reshape(-1), feat
